# Initial kernel scaffold; baseline (speedup 1.0000x reference)
import jax
import jax.numpy as jnp
from jax.experimental import pallas as pl


def kernel(x, edge_index, batch, mol_feats, Wg0, bg0, Wg1, bg1, Wg2, bg2, gamma_gc, beta_gc, Wm0, bm0, gamma_m, beta_m, Wm1, bm1, Wf0, bf0, Wf1, bf1, Wf2, bf2):
    raise NotImplementedError("write your pallas kernel here")


# speedup vs baseline: 12.7216x; 12.7216x over previous
import functools

import jax
import jax.numpy as jnp
from jax import lax
from jax.experimental import pallas as pl
from jax.experimental.pallas import tpu as pltpu
from jax.experimental.pallas import tpu_sc as plsc

N_NODES = 10000
N_PAD = 10240
N_EDGES = 320000
N_GRAPHS = 256
D_IN = 128
D0 = 64
D1 = 64
D2 = 128
D3 = 64
N_MOL = 200

NC = 2
NS = 16
NW = NC * NS
EPW = N_EDGES // NW
K = 80
NCHUNK = EPW // K
RPT = N_PAD // NS


@functools.lru_cache(maxsize=None)
def _sc_kernels():
    mesh = plsc.VectorSubcoreMesh(core_axis_name="c", subcore_axis_name="s",
                                  num_cores=NC, num_subcores=NS)

    @functools.partial(
        pl.kernel,
        out_type=jax.ShapeDtypeStruct((2 * N_PAD, 16), jnp.float32),
        mesh=mesh,
        compiler_params=pltpu.CompilerParams(use_tc_tiling_on_sc=False),
        scratch_types=[
            pltpu.VMEM((K,), jnp.int32),
            pltpu.VMEM((K, 16), jnp.float32),
            pltpu.VMEM_SHARED((N_PAD, 16), jnp.float32),
        ],
    )
    def deg_kernel(dst_hbm, ones_hbm, zeros_hbm, out_hbm, idx_d, ones_v, acc):
        cid = lax.axis_index("c")
        sid = lax.axis_index("s")
        wid = cid * NS + sid
        pltpu.sync_copy(ones_hbm, ones_v)
        pltpu.sync_copy(zeros_hbm.at[pl.ds(sid * RPT, RPT)],
                        acc.at[pl.ds(sid * RPT, RPT)])
        plsc.subcore_barrier()
        ebase = wid * EPW

        @pl.loop(0, NCHUNK)
        def _(i):
            b = ebase + i * K
            pltpu.sync_copy(dst_hbm.at[pl.ds(b, K)], idx_d)
            pltpu.sync_copy(ones_v, acc.at[idx_d], add=True)

        plsc.subcore_barrier()
        pltpu.sync_copy(acc.at[pl.ds(sid * RPT, RPT)],
                        out_hbm.at[pl.ds(cid * N_PAD + sid * RPT, RPT)])

    @functools.partial(
        pl.kernel,
        out_type=jax.ShapeDtypeStruct((2 * N_PAD, D0), jnp.float32),
        mesh=mesh,
        compiler_params=pltpu.CompilerParams(use_tc_tiling_on_sc=False),
        scratch_types=[
            pltpu.VMEM((K,), jnp.int32),
            pltpu.VMEM((K,), jnp.int32),
            pltpu.VMEM((K, D0), jnp.float32),
            pltpu.VMEM_SHARED((N_PAD, D0), jnp.float32),
            pltpu.SemaphoreType.DMA,
        ],
    )
    def spread_kernel(table_hbm, src_hbm, dst_hbm, zeros_hbm, out_hbm,
                      idx_s, idx_d, rows, acc, sem):
        cid = lax.axis_index("c")
        sid = lax.axis_index("s")
        wid = cid * NS + sid
        pltpu.sync_copy(zeros_hbm.at[pl.ds(sid * RPT, RPT)],
                        acc.at[pl.ds(sid * RPT, RPT)])
        plsc.subcore_barrier()
        ebase = wid * EPW

        @pl.loop(0, NCHUNK)
        def _(i):
            b = ebase + i * K
            pltpu.sync_copy(src_hbm.at[pl.ds(b, K)], idx_s)
            pltpu.sync_copy(dst_hbm.at[pl.ds(b, K)], idx_d)
            pltpu.async_copy(table_hbm.at[idx_s], rows, sem).wait()
            pltpu.sync_copy(rows, acc.at[idx_d], add=True)

        plsc.subcore_barrier()
        pltpu.sync_copy(acc.at[pl.ds(sid * RPT, RPT)],
                        out_hbm.at[pl.ds(cid * N_PAD + sid * RPT, RPT)])

    return deg_kernel, spread_kernel


def _dinv(deg_ref):
    d = deg_ref[0, :, 0:1] + deg_ref[1, :, 0:1] + 1.0
    return lax.rsqrt(d)


def _silu(z):
    return z * (1.0 / (1.0 + jnp.exp(-z)))


def _m0_body(x_ref, w_ref, deg_ref, out_ref):
    dinv = _dinv(deg_ref)
    out_ref[...] = jnp.dot(x_ref[...], w_ref[...],
                           preferred_element_type=jnp.float32) * dinv


def _m1_body(s_ref, t_ref, deg_ref, b_ref, gam_ref, bet_ref, w_ref, out_ref):
    dinv = _dinv(deg_ref)
    z = dinv * (s_ref[0] + s_ref[1] + t_ref[...]) + b_ref[...]
    mask = (lax.broadcasted_iota(jnp.int32, (N_PAD, 1), 0) < N_NODES
            ).astype(jnp.float32)
    zm = z * mask
    mean = jnp.sum(zm, axis=0, keepdims=True) / N_NODES
    var = jnp.sum(zm * zm, axis=0, keepdims=True) / N_NODES - mean * mean
    zbn = gam_ref[...] * (z - mean) * lax.rsqrt(var + 1e-5) + bet_ref[...]
    h = _silu(zbn)
    out_ref[...] = jnp.dot(h, w_ref[...],
                           preferred_element_type=jnp.float32) * dinv


def _m2_body(s_ref, t_ref, deg_ref, b_ref, w_ref, out_ref):
    dinv = _dinv(deg_ref)
    z = dinv * (s_ref[0] + s_ref[1] + t_ref[...]) + b_ref[...]
    h = _silu(z)
    out_ref[...] = jnp.dot(h, w_ref[...],
                           preferred_element_type=jnp.float32) * dinv


def _head_body(s_ref, t_ref, deg_ref, bg2_ref, batch_ref, mol_ref,
               wm0_ref, bm0_ref, gamm_ref, betm_ref, wm1_ref, bm1_ref,
               wf0a_ref, wf0b_ref, bf0_ref, wf1_ref, bf1_ref,
               wf2_ref, bf2_ref, out_ref):
    dinv = _dinv(deg_ref)
    z2 = _silu(dinv * (s_ref[0] + s_ref[1] + t_ref[...]) + bg2_ref[...])

    CH = 1024
    gids = lax.broadcasted_iota(jnp.int32, (N_GRAPHS, CH), 0)

    hg = jnp.zeros((N_GRAPHS, D1), jnp.float32)
    for i in range(N_PAD // CH):
        nbase = i * CH
        b = batch_ref[0:1, nbase:nbase + CH]
        col = lax.broadcasted_iota(jnp.int32, (N_GRAPHS, CH), 1) + nbase
        oh = jnp.where((gids == b) & (col < N_NODES), 1.0, 0.0)
        hg = hg + jnp.dot(oh, z2[nbase:nbase + CH, :],
                          preferred_element_type=jnp.float32)

    hm = jnp.dot(mol_ref[...], wm0_ref[...],
                 preferred_element_type=jnp.float32) + bm0_ref[...]
    mean = jnp.mean(hm, axis=0, keepdims=True)
    var = jnp.mean(hm * hm, axis=0, keepdims=True) - mean * mean
    hm = gamm_ref[...] * (hm - mean) * lax.rsqrt(var + 1e-5) + betm_ref[...]
    hm = _silu(hm)
    hm = _silu(jnp.dot(hm, wm1_ref[...],
                       preferred_element_type=jnp.float32) + bm1_ref[...])

    h = _silu(jnp.dot(hg, wf0a_ref[...], preferred_element_type=jnp.float32)
              + jnp.dot(hm, wf0b_ref[...], preferred_element_type=jnp.float32)
              + bf0_ref[...])
    h = _silu(jnp.dot(h, wf1_ref[...],
                      preferred_element_type=jnp.float32) + bf1_ref[...])
    out_ref[...] = jnp.dot(h, wf2_ref[...],
                           preferred_element_type=jnp.float32) + bf2_ref[...]


def _tc_call(body, out_shape, *args):
    return pl.pallas_call(
        body, out_shape=jax.ShapeDtypeStruct(out_shape, jnp.float32))(*args)


def kernel(x, edge_index, batch, mol_feats, Wg0, bg0, Wg1, bg1, Wg2, bg2,
           gamma_gc, beta_gc, Wm0, bm0, gamma_m, beta_m, Wm1, bm1,
           Wf0, bf0, Wf1, bf1, Wf2, bf2):
    f32 = jnp.float32
    src = edge_index[0].astype(jnp.int32)
    dst = edge_index[1].astype(jnp.int32)

    x_pad = jnp.zeros((N_PAD, D_IN), f32).at[:N_NODES].set(x)
    batch_pad = jnp.zeros((1, N_PAD), jnp.int32).at[0, :N_NODES].set(
        batch.astype(jnp.int32))
    zeros64 = jnp.zeros((N_PAD, D0), f32)
    zeros16 = jnp.zeros((N_PAD, 16), f32)
    ones16 = jnp.ones((K, 16), f32)
    mol_pad = jnp.zeros((N_GRAPHS, 256), f32).at[:, :N_MOL].set(mol_feats)
    Wm0_pad = jnp.zeros((256, D0), f32).at[:N_MOL].set(Wm0)
    Wf2_pad = jnp.zeros((D3, 128), f32).at[:, :1].set(Wf2)
    bf2_pad = jnp.zeros((128,), f32).at[:1].set(bf2)

    deg_kernel, spread_kernel = _sc_kernels()
    deg = deg_kernel(dst, ones16, zeros16).reshape(2, N_PAD, 16)

    h2_0 = _tc_call(_m0_body, (N_PAD, D0), x_pad, Wg0, deg)
    s0 = spread_kernel(h2_0, src, dst, zeros64).reshape(2, N_PAD, D0)
    h2_1 = _tc_call(_m1_body, (N_PAD, D0), s0, h2_0, deg,
                    bg0, gamma_gc, beta_gc, Wg1)
    s1 = spread_kernel(h2_1, src, dst, zeros64).reshape(2, N_PAD, D0)
    h2_2 = _tc_call(_m2_body, (N_PAD, D1), s1, h2_1, deg, bg1, Wg2)
    s2 = spread_kernel(h2_2, src, dst, zeros64).reshape(2, N_PAD, D1)

    out = _tc_call(_head_body, (N_GRAPHS, 128), s2, h2_2, deg, bg2,
                   batch_pad, mol_pad, Wm0_pad, bm0, gamma_m, beta_m,
                   Wm1, bm1, Wf0[:D1], Wf0[D1:], bf0, Wf1, bf1,
                   Wf2_pad, bf2_pad)
    return out[:, :1]


# speedup vs baseline: 14.7462x; 1.1591x over previous
import functools

import jax
import jax.numpy as jnp
from jax import lax
from jax.experimental import pallas as pl
from jax.experimental.pallas import tpu as pltpu
from jax.experimental.pallas import tpu_sc as plsc

N_NODES = 10000
N_PAD = 10240
N_EDGES = 320000
N_GRAPHS = 256
D_IN = 128
D0 = 64
D1 = 64
D2 = 128
D3 = 64
N_MOL = 200

NC = 2
NS = 16
NW = NC * NS
K = 128
CPW = 80
EPW = K * CPW
E_PAD = NW * EPW
DUMP = N_PAD - 8
RPT = N_PAD // NS


@functools.lru_cache(maxsize=None)
def _sc_kernels():
    mesh = plsc.VectorSubcoreMesh(core_axis_name="c", subcore_axis_name="s",
                                  num_cores=NC, num_subcores=NS)

    @functools.partial(
        pl.kernel,
        out_type=jax.ShapeDtypeStruct((2 * N_PAD, 16), jnp.float32),
        mesh=mesh,
        compiler_params=pltpu.CompilerParams(use_tc_tiling_on_sc=False),
        scratch_types=[
            pltpu.VMEM((CPW, K), jnp.int32),
            pltpu.VMEM((K, 16), jnp.float32),
            pltpu.VMEM_SHARED((N_PAD, 16), jnp.float32),
            pltpu.SemaphoreType.DMA,
        ],
    )
    def deg_kernel(dst_hbm, ones_hbm, zeros_hbm, out_hbm, dst_all, ones_v,
                   acc, sem):
        cid = lax.axis_index("c")
        sid = lax.axis_index("s")
        wid = cid * NS + sid
        pltpu.sync_copy(ones_hbm, ones_v)
        pltpu.sync_copy(dst_hbm.at[pl.ds(wid * CPW, CPW)], dst_all)
        pltpu.sync_copy(zeros_hbm.at[pl.ds(sid * RPT, RPT)],
                        acc.at[pl.ds(sid * RPT, RPT)])
        plsc.subcore_barrier()

        @pl.loop(0, CPW)
        def _(i):
            pltpu.async_copy(ones_v, acc.at[dst_all.at[i]], sem, add=True)

        @pl.loop(0, CPW)
        def _(i):
            pltpu.make_async_copy(ones_v, acc.at[dst_all.at[i]], sem).wait()

        plsc.subcore_barrier()
        pltpu.sync_copy(acc.at[pl.ds(sid * RPT, RPT)],
                        out_hbm.at[pl.ds(cid * N_PAD + sid * RPT, RPT)])

    @functools.partial(
        pl.kernel,
        out_type=jax.ShapeDtypeStruct((2 * N_PAD, D0), jnp.float32),
        mesh=mesh,
        compiler_params=pltpu.CompilerParams(use_tc_tiling_on_sc=False),
        scratch_types=[
            pltpu.VMEM((CPW, K), jnp.int32),
            pltpu.VMEM((CPW, K), jnp.int32),
            pltpu.VMEM((K, D0), jnp.float32),
            pltpu.VMEM((K, D0), jnp.float32),
            pltpu.VMEM_SHARED((N_PAD, D0), jnp.float32),
            pltpu.SemaphoreType.DMA,
            pltpu.SemaphoreType.DMA,
        ],
    )
    def spread_kernel(table_hbm, src_hbm, dst_hbm, zeros_hbm, out_hbm,
                      src_all, dst_all, rows0, rows1, acc, sem0, sem1):
        cid = lax.axis_index("c")
        sid = lax.axis_index("s")
        wid = cid * NS + sid
        pltpu.sync_copy(src_hbm.at[pl.ds(wid * CPW, CPW)], src_all)
        pltpu.sync_copy(dst_hbm.at[pl.ds(wid * CPW, CPW)], dst_all)
        pltpu.sync_copy(zeros_hbm.at[pl.ds(sid * RPT, RPT)],
                        acc.at[pl.ds(sid * RPT, RPT)])
        plsc.subcore_barrier()

        @pl.loop(0, CPW // 2)
        def _(j):
            i0 = 2 * j
            g0 = pltpu.async_copy(table_hbm.at[src_all.at[i0]], rows0, sem0)

            @pl.when(j > 0)
            def _():
                pltpu.sync_copy(rows1, acc.at[dst_all.at[i0 - 1]], add=True)

            g0.wait()
            g1 = pltpu.async_copy(table_hbm.at[src_all.at[i0 + 1]], rows1,
                                  sem1)
            pltpu.sync_copy(rows0, acc.at[dst_all.at[i0]], add=True)
            g1.wait()

        pltpu.sync_copy(rows1, acc.at[dst_all.at[CPW - 1]], add=True)
        plsc.subcore_barrier()
        pltpu.sync_copy(acc.at[pl.ds(sid * RPT, RPT)],
                        out_hbm.at[pl.ds(cid * N_PAD + sid * RPT, RPT)])

    return deg_kernel, spread_kernel


def _dinv(deg_ref):
    d = deg_ref[0, :, 0:1] + deg_ref[1, :, 0:1] + 1.0
    return lax.rsqrt(d)


def _silu(z):
    return z * (1.0 / (1.0 + jnp.exp(-z)))


def _m0_body(x_ref, w_ref, deg_ref, out_ref):
    dinv = _dinv(deg_ref)
    out_ref[...] = jnp.dot(x_ref[...], w_ref[...],
                           preferred_element_type=jnp.float32) * dinv


def _m1_body(s_ref, t_ref, deg_ref, b_ref, gam_ref, bet_ref, w_ref, out_ref):
    dinv = _dinv(deg_ref)
    z = dinv * (s_ref[0] + s_ref[1] + t_ref[...]) + b_ref[...]
    mask = (lax.broadcasted_iota(jnp.int32, (N_PAD, 1), 0) < N_NODES
            ).astype(jnp.float32)
    zm = z * mask
    mean = jnp.sum(zm, axis=0, keepdims=True) / N_NODES
    var = jnp.sum(zm * zm, axis=0, keepdims=True) / N_NODES - mean * mean
    zbn = gam_ref[...] * (z - mean) * lax.rsqrt(var + 1e-5) + bet_ref[...]
    h = _silu(zbn)
    out_ref[...] = jnp.dot(h, w_ref[...],
                           preferred_element_type=jnp.float32) * dinv


def _m2_body(s_ref, t_ref, deg_ref, b_ref, w_ref, out_ref):
    dinv = _dinv(deg_ref)
    z = dinv * (s_ref[0] + s_ref[1] + t_ref[...]) + b_ref[...]
    h = _silu(z)
    out_ref[...] = jnp.dot(h, w_ref[...],
                           preferred_element_type=jnp.float32) * dinv


def _head_body(s_ref, t_ref, deg_ref, bg2_ref, batch_ref, mol_ref,
               wm0_ref, bm0_ref, gamm_ref, betm_ref, wm1_ref, bm1_ref,
               wf0a_ref, wf0b_ref, bf0_ref, wf1_ref, bf1_ref,
               wf2_ref, bf2_ref, out_ref):
    dinv = _dinv(deg_ref)
    z2 = _silu(dinv * (s_ref[0] + s_ref[1] + t_ref[...]) + bg2_ref[...])

    CH = 1024
    gids = lax.broadcasted_iota(jnp.int32, (N_GRAPHS, CH), 0)

    hg = jnp.zeros((N_GRAPHS, D1), jnp.float32)
    for i in range(N_PAD // CH):
        nbase = i * CH
        b = batch_ref[0:1, nbase:nbase + CH]
        col = lax.broadcasted_iota(jnp.int32, (N_GRAPHS, CH), 1) + nbase
        oh = jnp.where((gids == b) & (col < N_NODES), 1.0, 0.0)
        hg = hg + jnp.dot(oh, z2[nbase:nbase + CH, :],
                          preferred_element_type=jnp.float32)

    hm = jnp.dot(mol_ref[...], wm0_ref[...],
                 preferred_element_type=jnp.float32) + bm0_ref[...]
    mean = jnp.mean(hm, axis=0, keepdims=True)
    var = jnp.mean(hm * hm, axis=0, keepdims=True) - mean * mean
    hm = gamm_ref[...] * (hm - mean) * lax.rsqrt(var + 1e-5) + betm_ref[...]
    hm = _silu(hm)
    hm = _silu(jnp.dot(hm, wm1_ref[...],
                       preferred_element_type=jnp.float32) + bm1_ref[...])

    h = _silu(jnp.dot(hg, wf0a_ref[...], preferred_element_type=jnp.float32)
              + jnp.dot(hm, wf0b_ref[...], preferred_element_type=jnp.float32)
              + bf0_ref[...])
    h = _silu(jnp.dot(h, wf1_ref[...],
                      preferred_element_type=jnp.float32) + bf1_ref[...])
    out_ref[...] = jnp.dot(h, wf2_ref[...],
                           preferred_element_type=jnp.float32) + bf2_ref[...]


def _tc_call(body, out_shape, *args):
    return pl.pallas_call(
        body, out_shape=jax.ShapeDtypeStruct(out_shape, jnp.float32))(*args)


def kernel(x, edge_index, batch, mol_feats, Wg0, bg0, Wg1, bg1, Wg2, bg2,
           gamma_gc, beta_gc, Wm0, bm0, gamma_m, beta_m, Wm1, bm1,
           Wf0, bf0, Wf1, bf1, Wf2, bf2):
    f32 = jnp.float32
    src = edge_index[0].astype(jnp.int32)
    dst = edge_index[1].astype(jnp.int32)
    src2d = jnp.zeros((E_PAD,), jnp.int32).at[:N_EDGES].set(src)
    src2d = src2d.reshape(NW * CPW, K)
    dst2d = jnp.full((E_PAD,), DUMP, jnp.int32).at[:N_EDGES].set(dst)
    dst2d = dst2d.reshape(NW * CPW, K)

    x_pad = jnp.zeros((N_PAD, D_IN), f32).at[:N_NODES].set(x)
    batch_pad = jnp.zeros((1, N_PAD), jnp.int32).at[0, :N_NODES].set(
        batch.astype(jnp.int32))
    zeros64 = jnp.zeros((N_PAD, D0), f32)
    zeros16 = jnp.zeros((N_PAD, 16), f32)
    ones16 = jnp.ones((K, 16), f32)
    mol_pad = jnp.zeros((N_GRAPHS, 256), f32).at[:, :N_MOL].set(mol_feats)
    Wm0_pad = jnp.zeros((256, D0), f32).at[:N_MOL].set(Wm0)
    Wf2_pad = jnp.zeros((D3, 128), f32).at[:, :1].set(Wf2)
    bf2_pad = jnp.zeros((128,), f32).at[:1].set(bf2)

    deg_kernel, spread_kernel = _sc_kernels()
    deg = deg_kernel(dst2d, ones16, zeros16).reshape(2, N_PAD, 16)

    h2_0 = _tc_call(_m0_body, (N_PAD, D0), x_pad, Wg0, deg)
    s0 = spread_kernel(h2_0, src2d, dst2d, zeros64).reshape(2, N_PAD, D0)
    h2_1 = _tc_call(_m1_body, (N_PAD, D0), s0, h2_0, deg,
                    bg0, gamma_gc, beta_gc, Wg1)
    s1 = spread_kernel(h2_1, src2d, dst2d, zeros64).reshape(2, N_PAD, D0)
    h2_2 = _tc_call(_m2_body, (N_PAD, D1), s1, h2_1, deg, bg1, Wg2)
    s2 = spread_kernel(h2_2, src2d, dst2d, zeros64).reshape(2, N_PAD, D1)

    out = _tc_call(_head_body, (N_GRAPHS, 128), s2, h2_2, deg, bg2,
                   batch_pad, mol_pad, Wm0_pad, bm0, gamma_m, beta_m,
                   Wm1, bm1, Wf0[:D1], Wf0[D1:], bf0, Wf1, bf1,
                   Wf2_pad, bf2_pad)
    return out[:, :1]
